# baseline (speedup 1.0000x reference)
import functools

import jax
import jax.numpy as jnp
from jax import lax
from jax.experimental import pallas as pl
from jax.experimental.pallas import tpu as pltpu
from jax.experimental.pallas import tpu_sc as plsc

_N = 50000
_E = 800000
_T = 3
_NC = 2
_NS = 16
_NW = _NC * _NS
_L = 16
_EC = _E // _NW
_EB = 128
_ECP = _EC + _EB
_SPAN = 256
_SHIFT = 8
_NB = -(-_N // _SPAN)
_NBP = 256
_NPAD = _NB * _SPAN
_BN = 2000
_KMAX = -(-_NB // _NW)
_CH = 1000


def _mesh():
    return plsc.VectorSubcoreMesh(
        core_axis_name="c", subcore_axis_name="s",
        num_cores=_NC, num_subcores=_NS)


def _partition(src, dst):
    @functools.partial(
        pl.kernel,
        out_type=[
            jax.ShapeDtypeStruct((_NW * _ECP,), jnp.int32),
            jax.ShapeDtypeStruct((_NW * _NBP,), jnp.int32),
        ],
        mesh=_mesh(),
        scratch_types=[
            pltpu.VMEM((_CH + _L,), jnp.int32),
            pltpu.VMEM((_CH + _L,), jnp.int32),
            pltpu.VMEM((_ECP,), jnp.int32),
            pltpu.VMEM((_NBP,), jnp.int32),
            pltpu.SMEM((_NBP,), jnp.int32),
        ],
        compiler_params=pltpu.CompilerParams(needs_layout_passes=False),
    )
    def k(src_hbm, dst_hbm, epk_hbm, ls_hbm,
          in_src, in_dst, spk_v, ls_v, cnt_s):
        wid = lax.axis_index("s") * _NC + lax.axis_index("c")
        base = pl.multiple_of(wid * _EC, 8)
        zero16 = jnp.zeros((_L,), jnp.int32)
        iota = lax.iota(jnp.int32, _L)
        masks = [iota == e for e in range(_L)]

        def zcnt(b, c):
            cnt_s[b] = 0
            return c
        lax.fori_loop(0, _NBP, zcnt, 0)

        def hist_chunk(ci, c):
            pltpu.sync_copy(
                dst_hbm.at[pl.ds(pl.multiple_of(base + ci * _CH, 8), _CH)],
                in_dst.at[pl.ds(0, _CH)])

            def hist(i, cc, ne=_L):
                bvec = lax.shift_right_logical(
                    in_dst[pl.ds(i * _L, _L)], _SHIFT)
                for e in range(ne):
                    be = bvec[e]
                    cnt_s[be] = cnt_s[be] + 1
                return cc
            lax.fori_loop(0, _CH // _L, hist, 0)
            hist(_CH // _L, 0, ne=_CH - (_CH // _L) * _L)
            return c
        lax.fori_loop(0, _EC // _CH, hist_chunk, 0)

        def pfx(b, run):
            c = cnt_s[b]
            cnt_s[b] = run
            plsc.store_scatter(ls_v, [jnp.full((_L,), b, jnp.int32)],
                               jnp.full((_L,), run, jnp.int32),
                               mask=masks[0])
            return run + c
        lax.fori_loop(0, _NBP, pfx, jnp.int32(0))

        def zs(i, c):
            spk_v[pl.ds(i * _L, _L)] = zero16
            return c
        lax.fori_loop(0, _ECP // _L, zs, 0)

        def place_chunk(ci, c):
            off = pl.multiple_of(base + ci * _CH, 8)
            pltpu.sync_copy(src_hbm.at[pl.ds(off, _CH)],
                            in_src.at[pl.ds(0, _CH)])
            pltpu.sync_copy(dst_hbm.at[pl.ds(off, _CH)],
                            in_dst.at[pl.ds(0, _CH)])

            def place(i, cc, ne=_L):
                svec = in_src[pl.ds(i * _L, _L)]
                dvec = in_dst[pl.ds(i * _L, _L)]
                bvec = lax.shift_right_logical(dvec, _SHIFT)
                pkvec = svec | lax.shift_left(dvec & (_SPAN - 1), 16)
                for e in range(ne):
                    be = bvec[e]
                    p = cnt_s[be]
                    cnt_s[be] = p + 1
                    pidx = jnp.full((_L,), p, jnp.int32)
                    plsc.store_scatter(spk_v, [pidx], pkvec, mask=masks[e])
                return cc
            lax.fori_loop(0, _CH // _L, place, 0)
            place(_CH // _L, 0, ne=_CH - (_CH // _L) * _L)
            return c
        lax.fori_loop(0, _EC // _CH, place_chunk, 0)

        obase = pl.multiple_of(wid * _ECP, 8)
        pltpu.sync_copy(spk_v, epk_hbm.at[pl.ds(obase, _ECP)])
        lsbase = pl.multiple_of(wid * _NBP, 8)
        pltpu.sync_copy(ls_v, ls_hbm.at[pl.ds(lsbase, _NBP)])

    return k(src, dst)


def _aggregate(ftel, er, epk, ls, hh, dd):
    hd = hh * dd
    tw = ftel.shape[1]
    nk = hd // _L
    dk = dd // _L

    @functools.partial(
        pl.kernel,
        out_type=[
            jax.ShapeDtypeStruct((_NPAD, hd), jnp.float32),
            jax.ShapeDtypeStruct((_NPAD, _L), jnp.float32),
        ],
        mesh=_mesh(),
        scratch_types=[
            pltpu.VMEM((_SPAN, hd), jnp.float32),
            pltpu.VMEM((_SPAN, _L), jnp.float32),
            pltpu.VMEM((_SPAN, 4), jnp.float32),
            pltpu.VMEM((_NW * _NBP,), jnp.int32),
            pltpu.VMEM((_EB, tw), jnp.float32),
            pltpu.VMEM((_EB,), jnp.int32),
            pltpu.VMEM((_EB,), jnp.int32),
            pltpu.SemaphoreType.DMA,
        ],
        compiler_params=pltpu.CompilerParams(needs_layout_passes=False),
    )
    def k(ftel_hbm, er_hbm, epk_hbm, ls_hbm, acc_hbm, den_hbm,
          acc_v, den_v, er_v, ls_v, rows_v, epk_v, sidx_v, sem):
        wid = lax.axis_index("s") * _NC + lax.axis_index("c")
        pltpu.sync_copy(ls_hbm, ls_v)
        iota = lax.iota(jnp.int32, _L)
        zf = jnp.zeros((_L,), jnp.float32)
        ohs = [(iota == h).astype(jnp.float32) for h in range(hh)]

        def bucket_body(kk, c0):
            b = wid + kk * _NW

            @pl.when(b < _NB)
            def _():
                node0 = pl.multiple_of(b * _SPAN, 8)
                bvec = jnp.full((_L,), b, jnp.int32)

                def zz(r, c):
                    for ci in range(nk):
                        acc_v[r, pl.ds(ci * _L, _L)] = zf
                    den_v[r, :] = zf
                    return c
                lax.fori_loop(0, _SPAN, zz, 0)

                pltpu.sync_copy(er_hbm.at[pl.ds(node0, _SPAN)], er_v)

                def st_body(st, c1):
                    stoff = jnp.full((_L,), st * _NBP, jnp.int32)
                    s0 = plsc.load_gather(ls_v, [stoff + bvec])[0]
                    s1 = plsc.load_gather(ls_v, [stoff + bvec + 1])[0]
                    astart = s0 & (-8)
                    nbat = (s1 - astart + (_EB - 1)) // _EB

                    def batch(j, c2):
                        off = astart + j * _EB
                        foff = pl.multiple_of(st * _ECP + off, 8)
                        pltpu.sync_copy(epk_hbm.at[pl.ds(foff, _EB)],
                                        epk_v)
                        for q in range(_EB // _L):
                            qs = pl.ds(q * _L, _L)
                            sidx_v[qs] = jnp.clip(epk_v[qs] & 0xFFFF,
                                                  0, _N - 1)
                        pltpu.async_copy(ftel_hbm.at[sidx_v], rows_v,
                                         sem).wait()
                        for i in range(_EB // _L):
                            g = off + i * _L + iota
                            valid = (g >= s0) & (g < s1)
                            pk16 = epk_v[pl.ds(i * _L, _L)]
                            loc = lax.shift_right_logical(pk16, 16)
                            svs = []
                            for h in range(hh):
                                elv = plsc.load_gather(
                                    rows_v,
                                    [i * _L + iota,
                                     jnp.full((_L,), hd + h, jnp.int32)])
                                erv = plsc.load_gather(
                                    er_v,
                                    [loc, jnp.full((_L,), h, jnp.int32)])
                                z = elv + erv
                                z = jnp.where(z >= 0, z, 0.2 * z)
                                svs.append(jnp.where(valid, jnp.exp(z), 0.0))
                            for e in range(_L):
                                le = loc[e]
                                ss = [svs[h][e] for h in range(hh)]
                                svec = zf
                                for h in range(hh):
                                    svec = svec + ss[h] * ohs[h]
                                den_v[le, :] = den_v[le, :] + svec
                                erow = i * _L + e
                                for ci in range(nk):
                                    sl = pl.ds(ci * _L, _L)
                                    acc_v[le, sl] = (acc_v[le, sl]
                                                     + rows_v[erow, sl]
                                                     * ss[ci // dk])
                        return c2
                    lax.fori_loop(0, nbat, batch, 0)
                    return c1
                lax.fori_loop(0, _NW, st_body, 0)

                pltpu.sync_copy(acc_v, acc_hbm.at[pl.ds(node0, _SPAN)])
                pltpu.sync_copy(den_v, den_hbm.at[pl.ds(node0, _SPAN)])
            return c0
        lax.fori_loop(0, _KMAX, bucket_body, 0)

    return k(ftel, er, epk, ls)


def _emit_dense(h, w_ref, al_ref, ar_ref, ftel_ref, er_ref):
    w = w_ref[...]
    ft = jnp.dot(h, w.T, preferred_element_type=jnp.float32)
    hh, dd = al_ref.shape
    bn = ft.shape[0]
    f3 = ft.reshape(bn, hh, dd)
    el = jnp.sum(f3 * al_ref[...][None], axis=-1)
    er = jnp.sum(f3 * ar_ref[...][None], axis=-1)
    tw = ftel_ref.shape[1]
    padw = tw - hh * dd - hh
    ftel_ref[...] = jnp.concatenate(
        [ft, el, jnp.zeros((bn, padw), jnp.float32)], axis=1)
    er_ref[...] = jnp.concatenate(
        [er, jnp.zeros((bn, 4 - hh), jnp.float32)], axis=1)


def _dense0_body(x_ref, w_ref, al_ref, ar_ref, ftel_ref, er_ref):
    _emit_dense(x_ref[...], w_ref, al_ref, ar_ref, ftel_ref, er_ref)


def _densemid_body(hh_prev, acc_ref, den_ref, b_ref, w_ref, al_ref, ar_ref,
                   ftel_ref, er_ref):
    a = acc_ref[...]
    bn, hdp = a.shape
    rep = hdp // hh_prev
    dh = den_ref[:, :hh_prev]
    dr = jnp.broadcast_to(dh[:, :, None], (bn, hh_prev, rep)).reshape(bn, hdp)
    v = jnp.where(dr > 0, a / dr, 0.0) + b_ref[...]
    h = jnp.where(v >= 0, v, 0.01 * v)
    _emit_dense(h, w_ref, al_ref, ar_ref, ftel_ref, er_ref)


def _dense_specs(hh, dd, hd, fin, tw):
    in_specs = [
        pl.BlockSpec((_BN, fin), lambda i: (i, 0)),
        pl.BlockSpec((hd, fin), lambda i: (0, 0)),
        pl.BlockSpec((hh, dd), lambda i: (0, 0)),
        pl.BlockSpec((hh, dd), lambda i: (0, 0)),
    ]
    out_specs = [
        pl.BlockSpec((_BN, tw), lambda i: (i, 0)),
        pl.BlockSpec((_BN, 4), lambda i: (i, 0)),
    ]
    out_shape = [
        jax.ShapeDtypeStruct((_N, tw), jnp.float32),
        jax.ShapeDtypeStruct((_NPAD, 4), jnp.float32),
    ]
    return in_specs, out_specs, out_shape


def _dense0(x, w, al, ar, tw):
    hh, dd = al.shape
    ins, outs, oshape = _dense_specs(hh, dd, w.shape[0], x.shape[1], tw)
    return pl.pallas_call(
        _dense0_body, grid=(_N // _BN,),
        in_specs=ins, out_specs=outs, out_shape=oshape,
    )(x, w, al, ar)


def _densemid(acc, den, bias, w, al, ar, hh_prev, tw):
    hh, dd = al.shape
    hdp = acc.shape[1]
    ins, outs, oshape = _dense_specs(hh, dd, w.shape[0], hdp, tw)
    ins = [
        pl.BlockSpec((_BN, hdp), lambda i: (i, 0)),
        pl.BlockSpec((_BN, _L), lambda i: (i, 0)),
        pl.BlockSpec((1, hdp), lambda i: (0, 0)),
    ] + ins[1:]
    return pl.pallas_call(
        functools.partial(_densemid_body, hh_prev), grid=(_N // _BN,),
        in_specs=ins, out_specs=outs, out_shape=oshape,
    )(acc, den, bias.reshape(1, -1), w, al, ar)


def _final_body(a0, d0, a1, d1, a2, d2, bo_ref, out_ref):
    o = jnp.zeros(out_ref.shape, jnp.float32)
    for t, (a_ref, d_ref) in enumerate(((a0, d0), (a1, d1), (a2, d2))):
        a = a_ref[...]
        d = d_ref[:, :1]
        o = o + jnp.where(d > 0, a / d, 0.0) + bo_ref[t, :][None, :]
    out_ref[...] = o * (1.0 / 3.0)


def _final(parts, bo):
    ispec = []
    for _ in range(_T):
        ispec.append(pl.BlockSpec((_BN, 32), lambda i: (i, 0)))
        ispec.append(pl.BlockSpec((_BN, _L), lambda i: (i, 0)))
    ispec.append(pl.BlockSpec((_T, 32), lambda i: (0, 0)))
    return pl.pallas_call(
        _final_body, grid=(_N // _BN,),
        in_specs=ispec,
        out_specs=pl.BlockSpec((_BN, 32), lambda i: (i, 0)),
        out_shape=jax.ShapeDtypeStruct((_N, 32), jnp.float32),
    )(*parts, bo)


def kernel(inputs, edge_index, emb0, emb1, emb2, W1, al1, ar1, b1,
           W2, al2, ar2, b2, Wo, alo, aro, bo):
    idx0 = inputs[:, 0].astype(jnp.int32)
    idx1 = inputs[:, 1].astype(jnp.int32)
    idx2 = inputs[:, 2].astype(jnp.int32)
    x = jnp.concatenate(
        [emb0[idx0], emb1[idx1], emb2[idx2], inputs[:, 3:]], axis=1)
    parts = []
    for t in range(_T):
        epk, ls = _partition(edge_index[t, 0], edge_index[t, 1])
        ftel, er = _dense0(x, W1[t], al1[t], ar1[t], 128)
        acc, den = _aggregate(ftel, er, epk, ls, 3, 32)
        ftel, er = _densemid(acc, den, b1[t], W2[t], al2[t], ar2[t], 3, 128)
        acc, den = _aggregate(ftel, er, epk, ls, 3, 32)
        ftel, er = _densemid(acc, den, b2[t], Wo[t], alo[t], aro[t], 3, 128)
        acc, den = _aggregate(ftel, er, epk, ls, 1, 32)
        parts += [acc, den]
    return _final(parts, bo)


# speedup vs baseline: 1.3138x; 1.3138x over previous
import functools

import jax
import jax.numpy as jnp
from jax import lax
from jax.experimental import pallas as pl
from jax.experimental.pallas import tpu as pltpu
from jax.experimental.pallas import tpu_sc as plsc

_N = 50000
_E = 800000
_T = 3
_NC = 2
_NS = 16
_NW = _NC * _NS
_L = 16
_EC = _E // _NW
_EB = 64
_ECP = _EC + _EB
_SPAN = 256
_SHIFT = 8
_NB = -(-_N // _SPAN)
_NBP = 256
_NPAD = _NB * _SPAN
_BN = 2000
_KMAX = -(-_NB // _NW)
_CH = 1000


def _mesh():
    return plsc.VectorSubcoreMesh(
        core_axis_name="c", subcore_axis_name="s",
        num_cores=_NC, num_subcores=_NS)


def _partition(src, dst):
    @functools.partial(
        pl.kernel,
        out_type=[
            jax.ShapeDtypeStruct((_NW * _ECP,), jnp.int32),
            jax.ShapeDtypeStruct((_NW * _NBP,), jnp.int32),
        ],
        mesh=_mesh(),
        scratch_types=[
            pltpu.VMEM((_CH + _L,), jnp.int32),
            pltpu.VMEM((_CH + _L,), jnp.int32),
            pltpu.VMEM((_ECP,), jnp.int32),
            pltpu.VMEM((_NBP,), jnp.int32),
            pltpu.SMEM((_NBP,), jnp.int32),
        ],
        compiler_params=pltpu.CompilerParams(needs_layout_passes=False),
    )
    def k(src_hbm, dst_hbm, epk_hbm, ls_hbm,
          in_src, in_dst, spk_v, ls_v, cnt_s):
        wid = lax.axis_index("s") * _NC + lax.axis_index("c")
        base = pl.multiple_of(wid * _EC, 8)
        zero16 = jnp.zeros((_L,), jnp.int32)
        iota = lax.iota(jnp.int32, _L)
        masks = [iota == e for e in range(_L)]

        def zcnt(b, c):
            cnt_s[b] = 0
            return c
        lax.fori_loop(0, _NBP, zcnt, 0)

        def hist_chunk(ci, c):
            pltpu.sync_copy(
                dst_hbm.at[pl.ds(pl.multiple_of(base + ci * _CH, 8), _CH)],
                in_dst.at[pl.ds(0, _CH)])

            def hist(i, cc, ne=_L):
                bvec = lax.shift_right_logical(
                    in_dst[pl.ds(i * _L, _L)], _SHIFT)
                for e in range(ne):
                    be = bvec[e]
                    cnt_s[be] = cnt_s[be] + 1
                return cc
            lax.fori_loop(0, _CH // _L, hist, 0)
            hist(_CH // _L, 0, ne=_CH - (_CH // _L) * _L)
            return c
        lax.fori_loop(0, _EC // _CH, hist_chunk, 0)

        def pfx(b, run):
            c = cnt_s[b]
            cnt_s[b] = run
            plsc.store_scatter(ls_v, [jnp.full((_L,), b, jnp.int32)],
                               jnp.full((_L,), run, jnp.int32),
                               mask=masks[0])
            return run + c
        lax.fori_loop(0, _NBP, pfx, jnp.int32(0))

        def zs(i, c):
            spk_v[pl.ds(i * _L, _L)] = zero16
            return c
        lax.fori_loop(0, _ECP // _L, zs, 0)

        def place_chunk(ci, c):
            off = pl.multiple_of(base + ci * _CH, 8)
            pltpu.sync_copy(src_hbm.at[pl.ds(off, _CH)],
                            in_src.at[pl.ds(0, _CH)])
            pltpu.sync_copy(dst_hbm.at[pl.ds(off, _CH)],
                            in_dst.at[pl.ds(0, _CH)])

            def place(i, cc, ne=_L):
                svec = in_src[pl.ds(i * _L, _L)]
                dvec = in_dst[pl.ds(i * _L, _L)]
                bvec = lax.shift_right_logical(dvec, _SHIFT)
                pkvec = svec | lax.shift_left(dvec & (_SPAN - 1), 16)
                for e in range(ne):
                    be = bvec[e]
                    p = cnt_s[be]
                    cnt_s[be] = p + 1
                    pidx = jnp.full((_L,), p, jnp.int32)
                    plsc.store_scatter(spk_v, [pidx], pkvec, mask=masks[e])
                return cc
            lax.fori_loop(0, _CH // _L, place, 0)
            place(_CH // _L, 0, ne=_CH - (_CH // _L) * _L)
            return c
        lax.fori_loop(0, _EC // _CH, place_chunk, 0)

        obase = pl.multiple_of(wid * _ECP, 8)
        pltpu.sync_copy(spk_v, epk_hbm.at[pl.ds(obase, _ECP)])
        lsbase = pl.multiple_of(wid * _NBP, 8)
        pltpu.sync_copy(ls_v, ls_hbm.at[pl.ds(lsbase, _NBP)])

    return k(src, dst)


def _aggregate(ftel, er, epk, ls, hh, dd):
    hd = hh * dd
    tw = ftel.shape[1]
    nk = hd // _L
    dk = dd // _L

    @functools.partial(
        pl.kernel,
        out_type=[
            jax.ShapeDtypeStruct((_NPAD, hd), jnp.float32),
            jax.ShapeDtypeStruct((_NPAD, _L), jnp.float32),
        ],
        mesh=_mesh(),
        scratch_types=[
            pltpu.VMEM((_SPAN, hd), jnp.float32),
            pltpu.VMEM((_SPAN, _L), jnp.float32),
            pltpu.VMEM((_SPAN, 4), jnp.float32),
            pltpu.VMEM((_NW * _NBP,), jnp.int32),
            pltpu.VMEM((_EB, tw), jnp.float32),
            pltpu.VMEM((_EB,), jnp.int32),
            pltpu.VMEM((_EB,), jnp.int32),
            pltpu.SemaphoreType.DMA,
        ],
        compiler_params=pltpu.CompilerParams(needs_layout_passes=False),
    )
    def k(ftel_hbm, er_hbm, epk_hbm, ls_hbm, acc_hbm, den_hbm,
          acc_v, den_v, er_v, ls_v, rows_v, epk_v, sidx_v, sem):
        wid = lax.axis_index("s") * _NC + lax.axis_index("c")
        pltpu.sync_copy(ls_hbm, ls_v)
        iota = lax.iota(jnp.int32, _L)
        zf = jnp.zeros((_L,), jnp.float32)
        ohs = [(iota == h).astype(jnp.float32) for h in range(hh)]

        def bucket_body(kk, c0):
            b = wid + kk * _NW

            @pl.when(b < _NB)
            def _():
                node0 = pl.multiple_of(b * _SPAN, 8)
                bvec = jnp.full((_L,), b, jnp.int32)

                def zz(r, c):
                    for ci in range(nk):
                        acc_v[r, pl.ds(ci * _L, _L)] = zf
                    den_v[r, :] = zf
                    return c
                lax.fori_loop(0, _SPAN, zz, 0)

                pltpu.sync_copy(er_hbm.at[pl.ds(node0, _SPAN)], er_v)

                def st_body(st, c1):
                    stoff = jnp.full((_L,), st * _NBP, jnp.int32)
                    s0 = plsc.load_gather(ls_v, [stoff + bvec])[0]
                    s1 = plsc.load_gather(ls_v, [stoff + bvec + 1])[0]
                    astart = s0 & (-8)
                    nbat = (s1 - astart + (_EB - 1)) // _EB

                    def batch(j, c2):
                        off = astart + j * _EB
                        foff = pl.multiple_of(st * _ECP + off, 8)
                        pltpu.sync_copy(epk_hbm.at[pl.ds(foff, _EB)],
                                        epk_v)
                        for q in range(_EB // _L):
                            qs = pl.ds(q * _L, _L)
                            sidx_v[qs] = jnp.clip(epk_v[qs] & 0xFFFF,
                                                  0, _N - 1)
                        pltpu.async_copy(ftel_hbm.at[sidx_v], rows_v,
                                         sem).wait()
                        for i in range(_EB // _L):
                          @pl.when((off + i * _L < s1)
                                   & (off + (i + 1) * _L > s0))
                          def _(i=i):
                            g = off + i * _L + iota
                            valid = (g >= s0) & (g < s1)
                            pk16 = epk_v[pl.ds(i * _L, _L)]
                            loc = lax.shift_right_logical(pk16, 16)
                            svs = []
                            for h in range(hh):
                                elv = plsc.load_gather(
                                    rows_v,
                                    [i * _L + iota,
                                     jnp.full((_L,), hd + h, jnp.int32)])
                                erv = plsc.load_gather(
                                    er_v,
                                    [loc, jnp.full((_L,), h, jnp.int32)])
                                z = elv + erv
                                z = jnp.where(z >= 0, z, 0.2 * z)
                                svs.append(jnp.where(valid, jnp.exp(z), 0.0))
                            for e in range(_L):
                                le = loc[e]
                                ss = [svs[h][e] for h in range(hh)]
                                svec = zf
                                for h in range(hh):
                                    svec = svec + ss[h] * ohs[h]
                                den_v[le, :] = den_v[le, :] + svec
                                erow = i * _L + e
                                for ci in range(nk):
                                    sl = pl.ds(ci * _L, _L)
                                    acc_v[le, sl] = (acc_v[le, sl]
                                                     + rows_v[erow, sl]
                                                     * ss[ci // dk])
                        return c2
                    lax.fori_loop(0, nbat, batch, 0)
                    return c1
                lax.fori_loop(0, _NW, st_body, 0)

                pltpu.sync_copy(acc_v, acc_hbm.at[pl.ds(node0, _SPAN)])
                pltpu.sync_copy(den_v, den_hbm.at[pl.ds(node0, _SPAN)])
            return c0
        lax.fori_loop(0, _KMAX, bucket_body, 0)

    return k(ftel, er, epk, ls)


def _emit_dense(h, w_ref, al_ref, ar_ref, ftel_ref, er_ref):
    w = w_ref[...]
    ft = jnp.dot(h, w.T, preferred_element_type=jnp.float32)
    hh, dd = al_ref.shape
    bn = ft.shape[0]
    f3 = ft.reshape(bn, hh, dd)
    el = jnp.sum(f3 * al_ref[...][None], axis=-1)
    er = jnp.sum(f3 * ar_ref[...][None], axis=-1)
    tw = ftel_ref.shape[1]
    padw = tw - hh * dd - hh
    ftel_ref[...] = jnp.concatenate(
        [ft, el, jnp.zeros((bn, padw), jnp.float32)], axis=1)
    er_ref[...] = jnp.concatenate(
        [er, jnp.zeros((bn, 4 - hh), jnp.float32)], axis=1)


def _dense0_body(x_ref, w_ref, al_ref, ar_ref, ftel_ref, er_ref):
    _emit_dense(x_ref[...], w_ref, al_ref, ar_ref, ftel_ref, er_ref)


def _densemid_body(hh_prev, acc_ref, den_ref, b_ref, w_ref, al_ref, ar_ref,
                   ftel_ref, er_ref):
    a = acc_ref[...]
    bn, hdp = a.shape
    rep = hdp // hh_prev
    dh = den_ref[:, :hh_prev]
    dr = jnp.broadcast_to(dh[:, :, None], (bn, hh_prev, rep)).reshape(bn, hdp)
    v = jnp.where(dr > 0, a / dr, 0.0) + b_ref[...]
    h = jnp.where(v >= 0, v, 0.01 * v)
    _emit_dense(h, w_ref, al_ref, ar_ref, ftel_ref, er_ref)


def _dense_specs(hh, dd, hd, fin, tw):
    in_specs = [
        pl.BlockSpec((_BN, fin), lambda i: (i, 0)),
        pl.BlockSpec((hd, fin), lambda i: (0, 0)),
        pl.BlockSpec((hh, dd), lambda i: (0, 0)),
        pl.BlockSpec((hh, dd), lambda i: (0, 0)),
    ]
    out_specs = [
        pl.BlockSpec((_BN, tw), lambda i: (i, 0)),
        pl.BlockSpec((_BN, 4), lambda i: (i, 0)),
    ]
    out_shape = [
        jax.ShapeDtypeStruct((_N, tw), jnp.float32),
        jax.ShapeDtypeStruct((_NPAD, 4), jnp.float32),
    ]
    return in_specs, out_specs, out_shape


def _dense0(x, w, al, ar, tw):
    hh, dd = al.shape
    ins, outs, oshape = _dense_specs(hh, dd, w.shape[0], x.shape[1], tw)
    return pl.pallas_call(
        _dense0_body, grid=(_N // _BN,),
        in_specs=ins, out_specs=outs, out_shape=oshape,
    )(x, w, al, ar)


def _densemid(acc, den, bias, w, al, ar, hh_prev, tw):
    hh, dd = al.shape
    hdp = acc.shape[1]
    ins, outs, oshape = _dense_specs(hh, dd, w.shape[0], hdp, tw)
    ins = [
        pl.BlockSpec((_BN, hdp), lambda i: (i, 0)),
        pl.BlockSpec((_BN, _L), lambda i: (i, 0)),
        pl.BlockSpec((1, hdp), lambda i: (0, 0)),
    ] + ins[1:]
    return pl.pallas_call(
        functools.partial(_densemid_body, hh_prev), grid=(_N // _BN,),
        in_specs=ins, out_specs=outs, out_shape=oshape,
    )(acc, den, bias.reshape(1, -1), w, al, ar)


def _final_body(a0, d0, a1, d1, a2, d2, bo_ref, out_ref):
    o = jnp.zeros(out_ref.shape, jnp.float32)
    for t, (a_ref, d_ref) in enumerate(((a0, d0), (a1, d1), (a2, d2))):
        a = a_ref[...]
        d = d_ref[:, :1]
        o = o + jnp.where(d > 0, a / d, 0.0) + bo_ref[t, :][None, :]
    out_ref[...] = o * (1.0 / 3.0)


def _final(parts, bo):
    ispec = []
    for _ in range(_T):
        ispec.append(pl.BlockSpec((_BN, 32), lambda i: (i, 0)))
        ispec.append(pl.BlockSpec((_BN, _L), lambda i: (i, 0)))
    ispec.append(pl.BlockSpec((_T, 32), lambda i: (0, 0)))
    return pl.pallas_call(
        _final_body, grid=(_N // _BN,),
        in_specs=ispec,
        out_specs=pl.BlockSpec((_BN, 32), lambda i: (i, 0)),
        out_shape=jax.ShapeDtypeStruct((_N, 32), jnp.float32),
    )(*parts, bo)


def kernel(inputs, edge_index, emb0, emb1, emb2, W1, al1, ar1, b1,
           W2, al2, ar2, b2, Wo, alo, aro, bo):
    idx0 = inputs[:, 0].astype(jnp.int32)
    idx1 = inputs[:, 1].astype(jnp.int32)
    idx2 = inputs[:, 2].astype(jnp.int32)
    x = jnp.concatenate(
        [emb0[idx0], emb1[idx1], emb2[idx2], inputs[:, 3:]], axis=1)
    parts = []
    for t in range(_T):
        epk, ls = _partition(edge_index[t, 0], edge_index[t, 1])
        ftel, er = _dense0(x, W1[t], al1[t], ar1[t], 128)
        acc, den = _aggregate(ftel, er, epk, ls, 3, 32)
        ftel, er = _densemid(acc, den, b1[t], W2[t], al2[t], ar2[t], 3, 128)
        acc, den = _aggregate(ftel, er, epk, ls, 3, 32)
        ftel, er = _densemid(acc, den, b2[t], Wo[t], alo[t], aro[t], 3, 128)
        acc, den = _aggregate(ftel, er, epk, ls, 1, 32)
        parts += [acc, den]
    return _final(parts, bo)
